# baseline (speedup 1.0000x reference)
import functools

import jax
import jax.numpy as jnp
from jax import lax
from jax.experimental import pallas as pl
from jax.experimental.pallas import tpu as pltpu
from jax.experimental.pallas import tpu_sc as plsc

EMBED = 64
PAIR = 2 * EMBED
LANE = 128
NWORKERS = 32
J = 2


def _lut_body(struct_ref, group_ref, w_ref, b_ref, gamma_ref, beta_ref,
              lut2_ref):
    s = struct_ref[...]
    g = group_ref[...]
    g7 = jnp.concatenate(
        [g[0:1], g[0:1], g[0:1], g[1:2], g[1:2], g[2:3], g[2:3]], axis=0)
    comb = jnp.concatenate([s, g7], axis=1)
    out = jnp.dot(comb, w_ref[...], preferred_element_type=jnp.float32)
    out = out + b_ref[...]
    mean = jnp.mean(out, axis=1, keepdims=True)
    var = jnp.mean((out - mean) ** 2, axis=1, keepdims=True)
    out = (out - mean) * lax.rsqrt(var + 1e-5)
    lut = out * gamma_ref[...] + beta_ref[...]
    r = lax.broadcasted_iota(jnp.int32, (49, 7), 0)
    j = lax.broadcasted_iota(jnp.int32, (49, 7), 1)
    ea = (r // 7 == j).astype(jnp.float32)
    eb = (r % 7 == j).astype(jnp.float32)
    left = jnp.dot(ea, lut, preferred_element_type=jnp.float32)
    right = jnp.dot(eb, lut, preferred_element_type=jnp.float32)
    lut2_ref[...] = jnp.concatenate([left, right], axis=1)


def _make_lut2(struct_table, group_table, W_fusion, b_fusion, gamma, beta):
    return pl.pallas_call(
        _lut_body,
        out_shape=jax.ShapeDtypeStruct((49, PAIR), jnp.float32),
    )(struct_table, group_table, W_fusion,
      b_fusion.reshape(1, EMBED), gamma.reshape(1, EMBED),
      beta.reshape(1, EMBED))


def _sc_lookup(lut2, cidx):
    nrow = cidx.shape[0]
    per_w = nrow // NWORKERS
    steps = per_w // (2 * J)
    mesh = plsc.VectorSubcoreMesh(core_axis_name="c", subcore_axis_name="s")

    @functools.partial(
        pl.kernel,
        mesh=mesh,
        out_type=jax.ShapeDtypeStruct((nrow, LANE, PAIR), jnp.float32),
        scratch_types=[
            pltpu.VMEM((per_w, LANE), jnp.int32),
            pltpu.VMEM((J, LANE, PAIR), jnp.float32),
            pltpu.VMEM((J, LANE, PAIR), jnp.float32),
            pltpu.VMEM_SHARED((49, PAIR), jnp.float32),
            pltpu.SemaphoreType.DMA,
            pltpu.SemaphoreType.DMA,
        ],
    )
    def k(lut_hbm, cidx_hbm, out_hbm, cidx_all, buf0, buf1, lut_sh,
          sem0, sem1):
        sid = lax.axis_index("s")
        wid = sid * 2 + lax.axis_index("c")
        base = wid * per_w
        bufs = (buf0, buf1)
        sems = (sem0, sem1)

        @pl.when(sid == 0)
        def _():
            pltpu.sync_copy(lut_hbm, lut_sh)

        pltpu.sync_copy(cidx_hbm.at[pl.ds(base, per_w)], cidx_all)
        plsc.subcore_barrier()

        def gather(b, c):
            for j in range(J):
                pltpu.async_copy(
                    lut_sh.at[cidx_all.at[c * J + j]], bufs[b].at[j], sems[b])

        def drain(b):
            for j in range(J):
                pltpu.make_async_copy(
                    lut_sh.at[cidx_all.at[j]], bufs[b].at[j], sems[b]).wait()

        def store(b, c):
            pltpu.sync_copy(bufs[b], out_hbm.at[pl.ds(base + c * J, J)])

        gather(0, 0)

        def step(t, _):
            c0 = 2 * t
            gather(1, c0 + 1)
            drain(0)
            store(0, c0)

            @pl.when(t < steps - 1)
            def _():
                gather(0, c0 + 2)

            drain(1)
            store(1, c0 + 1)
            return 0

        lax.fori_loop(0, steps, step, 0)

    return k(lut2, cidx)


def kernel(structure_indices, struct_table, group_table, W_fusion, b_fusion,
           gamma, beta):
    B, L = structure_indices.shape
    lut2 = _make_lut2(struct_table, group_table, W_fusion, b_fusion, gamma,
                      beta)
    idx = structure_indices.reshape(-1)
    cidx = (idx[0::2] * 7 + idx[1::2]).reshape(-1, LANE)
    out = _sc_lookup(lut2, cidx)
    return out.reshape(B, L, EMBED)


# speedup vs baseline: 2.5696x; 2.5696x over previous
import functools

import jax
import jax.numpy as jnp
from jax import lax
from jax.experimental import pallas as pl
from jax.experimental.pallas import tpu as pltpu
from jax.experimental.pallas import tpu_sc as plsc

EMBED = 64
PAIR = 2 * EMBED
NWORKERS = 32
J = 2


def _lut_body(struct_ref, group_ref, w_ref, b_ref, gamma_ref, beta_ref,
              lut2_ref):
    s = struct_ref[...]
    g = group_ref[...]
    g7 = jnp.concatenate(
        [g[0:1], g[0:1], g[0:1], g[1:2], g[1:2], g[2:3], g[2:3]], axis=0)
    comb = jnp.concatenate([s, g7], axis=1)
    out = jnp.dot(comb, w_ref[...], preferred_element_type=jnp.float32)
    out = out + b_ref[...]
    mean = jnp.mean(out, axis=1, keepdims=True)
    var = jnp.mean((out - mean) ** 2, axis=1, keepdims=True)
    out = (out - mean) * lax.rsqrt(var + 1e-5)
    lut = out * gamma_ref[...] + beta_ref[...]
    r = lax.broadcasted_iota(jnp.int32, (49, 7), 0)
    j = lax.broadcasted_iota(jnp.int32, (49, 7), 1)
    ea = (r // 7 == j).astype(jnp.float32)
    eb = (r % 7 == j).astype(jnp.float32)
    left = jnp.dot(ea, lut, preferred_element_type=jnp.float32)
    right = jnp.dot(eb, lut, preferred_element_type=jnp.float32)
    lut2_ref[...] = jnp.concatenate([left, right], axis=1)


def _make_lut2(struct_table, group_table, W_fusion, b_fusion, gamma, beta):
    return pl.pallas_call(
        _lut_body,
        out_shape=jax.ShapeDtypeStruct((49, PAIR), jnp.float32),
    )(struct_table, group_table, W_fusion,
      b_fusion.reshape(1, EMBED), gamma.reshape(1, EMBED),
      beta.reshape(1, EMBED))


def _pack_body(idx_ref, sel_ref, cidx_ref):
    idxf = idx_ref[...].astype(jnp.float32)
    c = jnp.dot(idxf, sel_ref[...], preferred_element_type=jnp.float32)
    cidx_ref[...] = c.astype(jnp.int32)


def _pack_indices(idx):
    B, L = idx.shape
    half = L // 2
    i = lax.broadcasted_iota(jnp.int32, (L, half), 0)
    k = lax.broadcasted_iota(jnp.int32, (L, half), 1)
    sel = (7 * (i == 2 * k) + (i == 2 * k + 1)).astype(jnp.float32)
    blk = 1024
    return pl.pallas_call(
        _pack_body,
        grid=(B // blk,),
        in_specs=[
            pl.BlockSpec((blk, L), lambda b: (b, 0)),
            pl.BlockSpec((L, half), lambda b: (0, 0)),
        ],
        out_specs=pl.BlockSpec((blk, half), lambda b: (b, 0)),
        out_shape=jax.ShapeDtypeStruct((B, half), jnp.int32),
    )(idx, sel)


def _sc_lookup(lut2, cidx, L):
    B, half = cidx.shape
    per_w = B // NWORKERS
    steps = per_w // (2 * J)
    mesh = plsc.VectorSubcoreMesh(core_axis_name="c", subcore_axis_name="s")

    @functools.partial(
        pl.kernel,
        mesh=mesh,
        out_type=jax.ShapeDtypeStruct((B, half, PAIR), jnp.float32),
        scratch_types=[
            pltpu.VMEM((per_w, half), jnp.int32),
            pltpu.VMEM((J, half, PAIR), jnp.float32),
            pltpu.VMEM((J, half, PAIR), jnp.float32),
            pltpu.VMEM_SHARED((49, PAIR), jnp.float32),
            pltpu.SemaphoreType.DMA,
            pltpu.SemaphoreType.DMA,
            pltpu.SemaphoreType.DMA,
            pltpu.SemaphoreType.DMA,
        ],
    )
    def k(lut_hbm, cidx_hbm, out_hbm, cidx_all, buf0, buf1, lut_sh,
          sem0, sem1, ssem0, ssem1):
        sid = lax.axis_index("s")
        wid = sid * 2 + lax.axis_index("c")
        base = wid * per_w
        bufs = (buf0, buf1)
        sems = (sem0, sem1)
        ssems = (ssem0, ssem1)

        @pl.when(sid == 0)
        def _():
            pltpu.sync_copy(lut_hbm, lut_sh)

        pltpu.sync_copy(cidx_hbm.at[pl.ds(base, per_w)], cidx_all)
        plsc.subcore_barrier()

        def gather(b, c):
            for j in range(J):
                pltpu.async_copy(
                    lut_sh.at[cidx_all.at[c * J + j]], bufs[b].at[j], sems[b])

        def drain(b):
            for j in range(J):
                pltpu.make_async_copy(
                    lut_sh.at[cidx_all.at[j]], bufs[b].at[j], sems[b]).wait()

        def store(b, c):
            pltpu.async_copy(bufs[b], out_hbm.at[pl.ds(base + c * J, J)],
                             ssems[b])

        def wait_store(b):
            pltpu.make_async_copy(bufs[b], out_hbm.at[pl.ds(base, J)],
                                  ssems[b]).wait()

        gather(0, 0)

        def step(t, _):
            c0 = 2 * t
            drain(0)
            store(0, c0)

            @pl.when(t > 0)
            def _():
                wait_store(1)

            gather(1, c0 + 1)
            drain(1)
            store(1, c0 + 1)
            wait_store(0)

            @pl.when(t < steps - 1)
            def _():
                gather(0, c0 + 2)

            return 0

        lax.fori_loop(0, steps, step, 0)
        wait_store(1)

    return k(lut2, cidx)


def kernel(structure_indices, struct_table, group_table, W_fusion, b_fusion,
           gamma, beta):
    B, L = structure_indices.shape
    lut2 = _make_lut2(struct_table, group_table, W_fusion, b_fusion, gamma,
                      beta)
    cidx = _pack_indices(structure_indices)
    out = _sc_lookup(lut2, cidx, L)
    return out.reshape(B, L, EMBED)


# speedup vs baseline: 2.5701x; 1.0002x over previous
import functools

import jax
import jax.numpy as jnp
from jax import lax
from jax.experimental import pallas as pl
from jax.experimental.pallas import tpu as pltpu
from jax.experimental.pallas import tpu_sc as plsc

EMBED = 64
PAIR = 2 * EMBED
NSUB = 16
NWORKERS = 32
J = 2


def _lut_body(struct_ref, group_ref, w_ref, b_ref, gamma_ref, beta_ref,
              lut2_ref):
    s = struct_ref[...]
    g = group_ref[...]
    g7 = jnp.concatenate(
        [g[0:1], g[0:1], g[0:1], g[1:2], g[1:2], g[2:3], g[2:3]], axis=0)
    comb = jnp.concatenate([s, g7], axis=1)
    out = jnp.dot(comb, w_ref[...], preferred_element_type=jnp.float32)
    out = out + b_ref[...]
    mean = jnp.mean(out, axis=1, keepdims=True)
    var = jnp.mean((out - mean) ** 2, axis=1, keepdims=True)
    out = (out - mean) * lax.rsqrt(var + 1e-5)
    lut = out * gamma_ref[...] + beta_ref[...]
    r = lax.broadcasted_iota(jnp.int32, (49, 7), 0)
    j = lax.broadcasted_iota(jnp.int32, (49, 7), 1)
    ea = (r // 7 == j).astype(jnp.float32)
    eb = (r % 7 == j).astype(jnp.float32)
    left = jnp.dot(ea, lut, preferred_element_type=jnp.float32)
    right = jnp.dot(eb, lut, preferred_element_type=jnp.float32)
    pair = jnp.concatenate([left, right], axis=1)
    lut2_ref[...] = jnp.tile(pair, (NSUB, 1))


def _make_lut2(struct_table, group_table, W_fusion, b_fusion, gamma, beta):
    return pl.pallas_call(
        _lut_body,
        out_shape=jax.ShapeDtypeStruct((NSUB * 49, PAIR), jnp.float32),
    )(struct_table, group_table, W_fusion,
      b_fusion.reshape(1, EMBED), gamma.reshape(1, EMBED),
      beta.reshape(1, EMBED))


def _pack_body(idx_ref, sel_ref, cidx_ref):
    idxf = idx_ref[...].astype(jnp.float32)
    c = jnp.dot(idxf, sel_ref[...], preferred_element_type=jnp.float32)
    cidx_ref[...] = c.astype(jnp.int32) + 49 * pl.program_id(0)


def _pack_indices(idx):
    B, L = idx.shape
    half = L // 2
    i = lax.broadcasted_iota(jnp.int32, (L, half), 0)
    k = lax.broadcasted_iota(jnp.int32, (L, half), 1)
    sel = (7 * (i == 2 * k) + (i == 2 * k + 1)).astype(jnp.float32)
    blk = B // NSUB
    return pl.pallas_call(
        _pack_body,
        grid=(B // blk,),
        in_specs=[
            pl.BlockSpec((blk, L), lambda b: (b, 0)),
            pl.BlockSpec((L, half), lambda b: (0, 0)),
        ],
        out_specs=pl.BlockSpec((blk, half), lambda b: (b, 0)),
        out_shape=jax.ShapeDtypeStruct((B, half), jnp.int32),
    )(idx, sel)


def _sc_lookup(lut2, cidx, L):
    B, half = cidx.shape
    per_w = B // NWORKERS
    steps = per_w // (2 * J)
    mesh = plsc.VectorSubcoreMesh(core_axis_name="c", subcore_axis_name="s")

    @functools.partial(
        pl.kernel,
        mesh=mesh,
        out_type=jax.ShapeDtypeStruct((B, half, PAIR), jnp.float32),
        scratch_types=[
            pltpu.VMEM((per_w, half), jnp.int32),
            pltpu.VMEM((J, half, PAIR), jnp.float32),
            pltpu.VMEM((J, half, PAIR), jnp.float32),
            pltpu.VMEM_SHARED((NSUB * 49, PAIR), jnp.float32),
            pltpu.SemaphoreType.DMA,
            pltpu.SemaphoreType.DMA,
            pltpu.SemaphoreType.DMA,
            pltpu.SemaphoreType.DMA,
        ],
    )
    def k(lut_hbm, cidx_hbm, out_hbm, cidx_all, buf0, buf1, lut_sh,
          sem0, sem1, ssem0, ssem1):
        sid = lax.axis_index("s")
        wid = sid * 2 + lax.axis_index("c")
        base = wid * per_w
        bufs = (buf0, buf1)
        sems = (sem0, sem1)
        ssems = (ssem0, ssem1)

        @pl.when(sid == 0)
        def _():
            pltpu.sync_copy(lut_hbm, lut_sh)

        pltpu.sync_copy(cidx_hbm.at[pl.ds(base, per_w)], cidx_all)
        plsc.subcore_barrier()

        def gather(b, c):
            for j in range(J):
                pltpu.async_copy(
                    lut_sh.at[cidx_all.at[c * J + j]], bufs[b].at[j], sems[b])

        def drain(b):
            for j in range(J):
                pltpu.make_async_copy(
                    lut_sh.at[cidx_all.at[j]], bufs[b].at[j], sems[b]).wait()

        def store(b, c):
            pltpu.async_copy(bufs[b], out_hbm.at[pl.ds(base + c * J, J)],
                             ssems[b])

        def wait_store(b):
            pltpu.make_async_copy(bufs[b], out_hbm.at[pl.ds(base, J)],
                                  ssems[b]).wait()

        gather(0, 0)

        def step(t, _):
            c0 = 2 * t
            drain(0)
            store(0, c0)

            @pl.when(t > 0)
            def _():
                wait_store(1)

            gather(1, c0 + 1)
            drain(1)
            store(1, c0 + 1)
            wait_store(0)

            @pl.when(t < steps - 1)
            def _():
                gather(0, c0 + 2)

            return 0

        lax.fori_loop(0, steps, step, 0)
        wait_store(1)

    return k(lut2, cidx)


def kernel(structure_indices, struct_table, group_table, W_fusion, b_fusion,
           gamma, beta):
    B, L = structure_indices.shape
    lut2 = _make_lut2(struct_table, group_table, W_fusion, b_fusion, gamma,
                      beta)
    cidx = _pack_indices(structure_indices)
    out = _sc_lookup(lut2, cidx, L)
    return out.reshape(B, L, EMBED)
